# Initial kernel scaffold; baseline (speedup 1.0000x reference)
import jax
import jax.numpy as jnp
from jax.experimental import pallas as pl


def kernel(X, vertex, edges, dege, degv, W):
    raise NotImplementedError("write your pallas kernel here")


# speedup vs baseline: 4.8859x; 4.8859x over previous
import functools

import jax
import jax.numpy as jnp
from jax import lax
from jax.experimental import pallas as pl
from jax.experimental.pallas import tpu as pltpu
from jax.experimental.pallas import tpu_sc as plsc

N = 50000
E = 25000
NNZ = 800000
CIN = 128
COUT = 128

NC = 2
NS = 16
NW = NC * NS
GPC = 4
CW = 16

PPS = NNZ // NS
K = 125
NH = 4
NB2 = PPS // (K * NH)
NBUF = 4

C = 128
E_T = 1664
E_P = NS * E_T
EC = E_T // C
V_T = 3200
N_P = NS * V_T
VC = V_T // C


def _matmul(X, W):
  def body(x_ref, w_ref, o_ref):
    o_ref[...] = jnp.dot(x_ref[...], w_ref[...],
                         preferred_element_type=jnp.float32)
  return pl.pallas_call(
      body,
      grid=(125,),
      in_specs=[
          pl.BlockSpec((400, CIN), lambda i: (i, 0)),
          pl.BlockSpec((CIN, COUT), lambda i: (0, 0)),
      ],
      out_specs=pl.BlockSpec((400, COUT), lambda i: (i, 0)),
      out_shape=jax.ShapeDtypeStruct((N, COUT), jnp.float32),
  )(X, W)


def _sc_conv(xp0, xp1, xp2, xp3, vtxc, vtxu, edg, dege16, degv16):
  mesh = plsc.VectorSubcoreMesh(core_axis_name="c", subcore_axis_name="s")

  @functools.partial(
      pl.kernel,
      out_type=(
          jax.ShapeDtypeStruct((GPC, NC, N_P, CW), jnp.float32),
          jax.ShapeDtypeStruct((NC, E_P, CW), jnp.float32),
      ),
      mesh=mesh,
      compiler_params=pltpu.CompilerParams(use_tc_tiling_on_sc=False),
      scratch_types=[
          pltpu.VMEM((NB2, K), jnp.int32),
          pltpu.VMEM((NB2, K), jnp.int32),
          [pltpu.VMEM((K, CW), jnp.float32) for _ in range(NBUF)],
          pltpu.VMEM((K, CW), jnp.float32),
          pltpu.VMEM((C, CW), jnp.float32),
          pltpu.VMEM((C, CW), jnp.float32),
          pltpu.VMEM((C, CW), jnp.float32),
          pltpu.VMEM_SHARED((E_P, CW), jnp.float32),
          pltpu.VMEM_SHARED((N_P, CW), jnp.float32),
          [pltpu.SemaphoreType.DMA for _ in range(NBUF)],
          [pltpu.SemaphoreType.DMA for _ in range(NBUF)],
          pltpu.SemaphoreType.DMA,
      ],
  )
  def conv(xp0_h, xp1_h, xp2_h, xp3_h, vtxc_h, vtxu_h, edg_h, dege_h, degv_h,
           out_h, scale_h, gidx, sidx, bufs, ones_b, ta, tb, zb,
           xe_s, xv_s, gsem, asem, csem):
    cid = lax.axis_index("c")
    sid = lax.axis_index("s")
    xp_refs = (xp0_h, xp1_h, xp2_h, xp3_h)

    for q in range(C):
      zb[q, :] = jnp.zeros((CW,), jnp.float32)
    for q in range(K):
      ones_b[q, :] = jnp.full((CW,), 1.0, jnp.float32)

    eb = sid * E_T
    vb = sid * V_T

    def zero_xe():
      for k in range(EC):
        pltpu.sync_copy(zb, xe_s.at[pl.ds(eb + k * C, C)])

    def zero_xv():
      for k in range(VC):
        pltpu.sync_copy(zb, xv_s.at[pl.ds(vb + k * C, C)])

    zero_xe()
    zero_xv()
    plsc.subcore_barrier()

    def _pipe(src_fn, dst_fn, count):
      for p in range(NBUF):
        pltpu.async_copy(src_fn(p), bufs[p], gsem[p])

      def one(b, p):
        pltpu.make_async_copy(src_fn(b), bufs[p], gsem[p]).wait()
        pltpu.async_copy(bufs[p], dst_fn(b), asem[p], add=True)
        if count:
          pltpu.async_copy(ones_b, xv_s.at[sidx.at[b]], csem, add=True)
          pltpu.make_async_copy(ones_b, xv_s.at[sidx.at[b]], csem).wait()
        pltpu.make_async_copy(bufs[p], dst_fn(b), asem[p]).wait()

      def step(i, c):
        for p in range(NBUF):
          b = i * NBUF + p
          one(b, p)
          pltpu.async_copy(src_fn(b + NBUF), bufs[p], gsem[p])
        return c

      lax.fori_loop(0, NB2 // NBUF - 1, step, 0)
      for p in range(NBUF):
        one(NB2 - NBUF + p, p)

    def phase_b(j, count):
      for h in range(NH):
        pltpu.sync_copy(vtxc_h.at[cid, sid, h], gidx)
        pltpu.sync_copy(edg_h.at[sid, h], sidx)
        _pipe(lambda b: xp_refs[j].at[gidx.at[b]],
              lambda b: xe_s.at[sidx.at[b]],
              count=count)

    def phase_c():
      for h in range(NH):
        pltpu.sync_copy(edg_h.at[sid, h], gidx)
        pltpu.sync_copy(vtxu_h.at[sid, h], sidx)
        _pipe(lambda b: xe_s.at[gidx.at[b]],
              lambda b: xv_s.at[sidx.at[b]],
              count=False)

    for j in range(GPC):
      phase_b(j, count=(j == 0))
      plsc.subcore_barrier()

      if j == 0:
        def srow(k, c):
          r = eb + k * C
          pltpu.sync_copy(xv_s.at[pl.ds(r, C)], ta)
          pltpu.sync_copy(dege_h.at[pl.ds(r, C)], tb)
          for q in range(C):
            ta[q, :] = tb[q, :] / jnp.maximum(ta[q, :], 1.0)
          pltpu.sync_copy(ta, scale_h.at[cid, pl.ds(r, C)])
          pltpu.sync_copy(zb, xv_s.at[pl.ds(r, C)])
          return c
        lax.fori_loop(0, EC, srow, 0)
        plsc.subcore_barrier()

      def erow(k, c):
        r = eb + k * C
        pltpu.sync_copy(xe_s.at[pl.ds(r, C)], ta)
        pltpu.sync_copy(scale_h.at[cid, pl.ds(r, C)], tb)
        for q in range(C):
          ta[q, :] = ta[q, :] * tb[q, :]
        pltpu.sync_copy(ta, xe_s.at[pl.ds(r, C)])
        return c
      lax.fori_loop(0, EC, erow, 0)
      plsc.subcore_barrier()

      phase_c()
      plsc.subcore_barrier()

      def vrow(k, c):
        r = vb + k * C
        pltpu.sync_copy(xv_s.at[pl.ds(r, C)], ta)
        pltpu.sync_copy(degv_h.at[pl.ds(r, C)], tb)
        for q in range(C):
          ta[q, :] = ta[q, :] * tb[q, :]
        pltpu.sync_copy(ta, out_h.at[j, cid, pl.ds(r, C)])
        return c
      lax.fori_loop(0, VC, vrow, 0)

      if j < GPC - 1:
        zero_xe()
        zero_xv()
        plsc.subcore_barrier()

  return conv(xp0, xp1, xp2, xp3, vtxc, vtxu, edg, dege16, degv16)


@jax.jit
def kernel(X, vertex, edges, dege, degv, W):
  Xp = _matmul(X, W)
  xpg = Xp.reshape(N, NC, GPC, CW).transpose(2, 1, 0, 3)
  xp_j = [xpg[j].reshape(NC * N, CW) for j in range(GPC)]

  vtxu = vertex.reshape(NS, NH, NB2, K)
  vtxc = vtxu[None] + (jnp.arange(NC, dtype=jnp.int32) * N)[:, None, None,
                                                            None, None]
  edg = edges.reshape(NS, NH, NB2, K)

  dege16 = jnp.zeros((E_P, CW), jnp.float32).at[:E].set(
      jnp.broadcast_to(dege, (E, CW)))
  degv16 = jnp.zeros((N_P, CW), jnp.float32).at[:N].set(
      jnp.broadcast_to(degv, (N, CW)))

  out, _ = _sc_conv(xp_j[0], xp_j[1], xp_j[2], xp_j[3], vtxc, vtxu, edg,
                    dege16, degv16)
  return out[:, :, :N, :].transpose(2, 1, 0, 3).reshape(N, COUT)
